# Initial kernel scaffold; baseline (speedup 1.0000x reference)
import functools
import jax
import jax.numpy as jnp
from jax import lax
from jax.experimental import pallas as pl
from jax.experimental.pallas import tpu as pltpu
from jax.experimental.pallas import tpu_sc as plsc

NP = 10000
NC = 10000
E = 160000
D = 256

NCORES = 2
NSUB = 16
NW = NCORES * NSUB
C1 = E // NW
C4 = E // NSUB
HALF = NC // NCORES
TRASH = HALF

_mesh = plsc.VectorSubcoreMesh(core_axis_name="c", subcore_axis_name="s")

_f32 = jnp.float32
_i32 = jnp.int32


def _iota16():
    return lax.broadcasted_iota(_i32, (16,), 0)


def _sck1_body(dst_h, ew_h, p_out, acc_c, acc_1, acc_2,
               dstb2, ewb, onesb, ew2b, zbuf):
    c = lax.axis_index("c")
    s = lax.axis_index("s")
    wid = s * NCORES + c
    eb = pl.multiple_of(wid * C1, 8)

    z16f = jnp.zeros((16,), _f32)
    z16i = jnp.zeros((16,), _i32)

    @pl.when(s == 0)
    def _():
        def zb(i, _):
            zbuf[pl.ds(i * 16, 16)] = z16f
            return 0
        lax.fori_loop(0, NC // 16, zb, 0)
        pltpu.sync_copy(zbuf, acc_c)
        pltpu.sync_copy(zbuf, acc_1)
        pltpu.sync_copy(zbuf, acc_2)

    def fill(i, _):
        idx = i * 16 + _iota16()
        onesb[pl.ds(i * 16, 16)] = jnp.where(idx < C1, 1.0, 0.0).astype(_f32)
        return 0
    lax.fori_loop(0, 320, fill, 0)

    def ztail(i, _):
        ewb[pl.ds(C1 + i * 16, 16)] = z16f
        return 0
    lax.fori_loop(0, 120 // 16, ztail, 0)
    pltpu.sync_copy(ew_h.at[pl.ds(eb, C1)], ewb.at[pl.ds(0, C1)])

    def sq(i, _):
        v = ewb[pl.ds(i * 16, 16)]
        ew2b[pl.ds(i * 16, 16)] = v * v
        return 0
    lax.fori_loop(0, 320, sq, 0)

    for i in range(8):
        dstb2[39, pl.ds(i * 16, 16)] = z16i

    def ldidx(j, _):
        off = pl.multiple_of(j * 128, 128)
        pltpu.sync_copy(dst_h.at[pl.ds(eb + off, 128)], dstb2.at[j])
        return 0
    lax.fori_loop(0, 39, ldidx, 0)
    pltpu.sync_copy(dst_h.at[pl.ds(eb + 4992, 8)], dstb2.at[39, pl.ds(0, 8)])

    plsc.subcore_barrier()

    def scat(j, _):
        off = pl.multiple_of(j * 128, 128)
        idx = dstb2.at[j]
        pltpu.sync_copy(onesb.at[pl.ds(off, 128)], acc_c.at[idx], add=True)
        pltpu.sync_copy(ewb.at[pl.ds(off, 128)], acc_1.at[idx], add=True)
        pltpu.sync_copy(ew2b.at[pl.ds(off, 128)], acc_2.at[idx], add=True)
        return 0
    lax.fori_loop(0, 40, scat, 0)

    plsc.subcore_barrier()

    @pl.when(s == 0)
    def _():
        pltpu.sync_copy(acc_c, p_out.at[c, 0])

    @pl.when(s == 1)
    def _():
        pltpu.sync_copy(acc_1, p_out.at[c, 1])

    @pl.when(s == 2)
    def _():
        pltpu.sync_copy(acc_2, p_out.at[c, 2])


_sck1 = functools.partial(
    pl.kernel,
    out_type=jax.ShapeDtypeStruct((NCORES, 3, NC), _f32),
    mesh=_mesh,
    scratch_types=[
        pltpu.VMEM_SHARED((NC,), _f32),
        pltpu.VMEM_SHARED((NC,), _f32),
        pltpu.VMEM_SHARED((NC,), _f32),
        pltpu.VMEM((40, 128), _i32),
        pltpu.VMEM((5120,), _f32),
        pltpu.VMEM((5120,), _f32),
        pltpu.VMEM((5120,), _f32),
        pltpu.VMEM((NC,), _f32),
    ],
)(_sck1_body)


def _sck2_body(a_h, b_h, mean_h, inv_h, src_h, dst_h,
               sa_h, sb_h, me_h, iv_h,
               srcb, dstb, bufA, bufB, bufm, bufi, s0, s1, s2, s3):
    c = lax.axis_index("c")
    s = lax.axis_index("s")
    wid = s * NCORES + c
    eb = pl.multiple_of(wid * C1, 8)

    pltpu.sync_copy(src_h.at[pl.ds(eb, C1)], srcb.at[pl.ds(0, C1)])
    pltpu.sync_copy(dst_h.at[pl.ds(eb, C1)], dstb.at[pl.ds(0, C1)])

    def blk(j, _):
        off = pl.multiple_of(jnp.minimum(j * 128, C1 - 128), 8)
        si = srcb.at[pl.ds(off, 128)]
        di = dstb.at[pl.ds(off, 128)]
        d1 = pltpu.async_copy(a_h.at[si], bufA, s0)
        d2 = pltpu.async_copy(b_h.at[di], bufB, s1)
        d3 = pltpu.async_copy(mean_h.at[di], bufm, s2)
        d4 = pltpu.async_copy(inv_h.at[di], bufi, s3)
        d1.wait()
        d2.wait()
        d3.wait()
        d4.wait()
        base = pl.multiple_of(eb + off, 8)
        pltpu.sync_copy(bufA, sa_h.at[pl.ds(base, 128)])
        pltpu.sync_copy(bufB, sb_h.at[pl.ds(base, 128)])
        pltpu.sync_copy(bufm, me_h.at[pl.ds(base, 128)])
        pltpu.sync_copy(bufi, iv_h.at[pl.ds(base, 128)])
        return 0
    lax.fori_loop(0, 40, blk, 0)


_sck2 = functools.partial(
    pl.kernel,
    out_type=(
        jax.ShapeDtypeStruct((E, D), _f32),
        jax.ShapeDtypeStruct((E, D), _f32),
        jax.ShapeDtypeStruct((E,), _f32),
        jax.ShapeDtypeStruct((E,), _f32),
    ),
    mesh=_mesh,
    scratch_types=[
        pltpu.VMEM((5120,), _i32),
        pltpu.VMEM((5120,), _i32),
        pltpu.VMEM((128, D), _f32),
        pltpu.VMEM((128, D), _f32),
        pltpu.VMEM((128,), _f32),
        pltpu.VMEM((128,), _f32),
        pltpu.SemaphoreType.DMA,
        pltpu.SemaphoreType.DMA,
        pltpu.SemaphoreType.DMA,
        pltpu.SemaphoreType.DMA,
    ],
)(_sck2_body)


def _sck4_body(g_h, src_h, dst_h, ex_h, m_h, den_h,
               accM, accD, srcb, dstb, exb, bufG, idxw, exm, zrow, zrowD, sem):
    c = lax.axis_index("c")
    s = lax.axis_index("s")
    teb = pl.multiple_of(s * C4, 8)
    lo = c * HALF

    z16f = jnp.zeros((16,), _f32)
    z16i = jnp.zeros((16,), _i32)

    def zr(i, _):
        for k in range(16):
            zrow[i, pl.ds(k * 16, 16)] = z16f
        return 0
    lax.fori_loop(0, 128, zr, 0)

    def zd(i, _):
        zrowD[pl.ds(i * 16, 16)] = z16f
        return 0
    lax.fori_loop(0, 20, zd, 0)

    rs = pl.multiple_of(s * 320, 8)
    pltpu.sync_copy(zrow, accM.at[pl.ds(rs, 128)])
    pltpu.sync_copy(zrow, accM.at[pl.ds(rs + 128, 128)])
    pltpu.sync_copy(zrow.at[pl.ds(0, 64)], accM.at[pl.ds(rs + 256, 64)])
    pltpu.sync_copy(zrowD, accD.at[pl.ds(rs, 320)])

    plsc.subcore_barrier()

    def zed(i, _):
        srcb[pl.ds(C4 + i * 16, 16)] = z16i
        dstb[pl.ds(C4 + i * 16, 16)] = z16i
        exb[pl.ds(C4 + i * 16, 16)] = z16f
        return 0
    lax.fori_loop(0, 112 // 16, zed, 0)
    pltpu.sync_copy(src_h.at[pl.ds(teb, C4)], srcb.at[pl.ds(0, C4)])
    pltpu.sync_copy(dst_h.at[pl.ds(teb, C4)], dstb.at[pl.ds(0, C4)])
    pltpu.sync_copy(ex_h.at[pl.ds(teb, C4)], exb.at[pl.ds(0, C4)])

    def blk(j, _):
        off = pl.multiple_of(j * 128, 128)
        gd = pltpu.async_copy(g_h.at[srcb.at[pl.ds(off, 128)]], bufG, sem)
        for k in range(8):
            dv = dstb[pl.ds(off + k * 16, 16)]
            ev = exb[pl.ds(off + k * 16, 16)]
            dl = dv - lo
            valid = (dl >= 0) & (dl < HALF)
            dl = jnp.where(valid, dl, TRASH)
            ev = jnp.where(valid, ev, 0.0)
            idxw[0, pl.ds(k * 16, 16)] = dl
            exm[pl.ds(k * 16, 16)] = ev
        gd.wait()

        def scale(r, _):
            bex = plsc.load_gather(exm, [jnp.full((16,), r, _i32)])
            for k in range(16):
                bufG[r, pl.ds(k * 16, 16)] = bufG[r, pl.ds(k * 16, 16)] * bex
            return 0
        lax.fori_loop(0, 128, scale, 0)

        pltpu.sync_copy(bufG, accM.at[idxw.at[0]], add=True)
        pltpu.sync_copy(exm, accD.at[idxw.at[0]], add=True)
        return 0
    lax.fori_loop(0, 79, blk, 0)

    plsc.subcore_barrier()

    ws = pl.multiple_of(s * 312, 8)
    ob = pl.multiple_of(c * HALF + s * 312, 8)
    pltpu.sync_copy(accM.at[pl.ds(ws, 312)], m_h.at[pl.ds(ob, 312)])
    pltpu.sync_copy(accD.at[pl.ds(ws, 312)], den_h.at[pl.ds(ob, 312)])

    @pl.when(s == 0)
    def _():
        tb = pl.multiple_of(c * HALF + 4992, 8)
        pltpu.sync_copy(accM.at[pl.ds(4992, 8)], m_h.at[pl.ds(tb, 8)])
        pltpu.sync_copy(accD.at[pl.ds(4992, 8)], den_h.at[pl.ds(tb, 8)])


_sck4 = functools.partial(
    pl.kernel,
    out_type=(
        jax.ShapeDtypeStruct((NC, D), _f32),
        jax.ShapeDtypeStruct((NC,), _f32),
    ),
    mesh=_mesh,
    scratch_types=[
        pltpu.VMEM_SHARED((5120, D), _f32),
        pltpu.VMEM_SHARED((5120,), _f32),
        pltpu.VMEM((10112,), _i32),
        pltpu.VMEM((10112,), _i32),
        pltpu.VMEM((10112,), _f32),
        pltpu.VMEM((128, D), _f32),
        pltpu.VMEM((1, 128), _i32),
        pltpu.VMEM((128,), _f32),
        pltpu.VMEM((128, D), _f32),
        pltpu.VMEM((320,), _f32),
        pltpu.SemaphoreType.DMA,
    ],
)(_sck4_body)


def _tck2a_body(ph, cf, w1p, w1c, wg, wp, b1, bg, bp, a_o, b_o, g_o):
    x = ph[...]
    a_o[...] = jnp.dot(x, w1p[...], preferred_element_type=_f32)
    b_o[...] = jnp.dot(cf[...], w1c[...], preferred_element_type=_f32) + b1[...]
    gate = jax.nn.sigmoid(jnp.dot(x, wg[...], preferred_element_type=_f32) + bg[...])
    g_o[...] = gate * (jnp.dot(x, wp[...], preferred_element_type=_f32) + bp[...])


def _tck2b_body(p_ref, mean_o, inv_o):
    p = p_ref[...]
    cnt = p[0, 0] + p[1, 0]
    s1 = p[0, 1] + p[1, 1]
    s2 = p[0, 2] + p[1, 2]
    cntc = jnp.maximum(cnt, 1.0)
    mean = s1 / cntc
    var = jnp.maximum(s2 / cntc - mean * mean, 0.0)
    mean_o[...] = mean.reshape(1, NC)
    inv_o[...] = (1.0 / (jnp.sqrt(var + 1e-6) + 1e-6)).reshape(1, NC)


def _tck3_body(sa, sb, ew, me, iv, w2, b2, al, ex_o):
    t = jnp.tanh(sa[...] + sb[...])
    att = jnp.sum(t * w2[...], axis=1, keepdims=True)
    att = att + b2[...] + al[...] * (ew[...] - me[...]) * iv[...]
    ex_o[...] = jnp.exp(att)


def _tck5_body(m, den, out):
    out[...] = m[...] / (den[...] + 1e-16)


def kernel(pathway_h, cell_feat, edge_index, edge_weight,
           W_proj, b_proj, W_a1, b_a1, W_a2, b_a2, W_g, b_g, alpha_prior):
    src = edge_index[0].astype(_i32)
    dst = edge_index[1].astype(_i32)
    ew = edge_weight.astype(_f32)

    P = _sck1(dst, ew)

    R = 500
    A, B, G = pl.pallas_call(
        _tck2a_body,
        grid=(NP // R,),
        in_specs=[
            pl.BlockSpec((R, D), lambda i: (i, 0)),
            pl.BlockSpec((R, D), lambda i: (i, 0)),
            pl.BlockSpec((D, D), lambda i: (0, 0)),
            pl.BlockSpec((D, D), lambda i: (0, 0)),
            pl.BlockSpec((D, D), lambda i: (0, 0)),
            pl.BlockSpec((D, D), lambda i: (0, 0)),
            pl.BlockSpec((1, D), lambda i: (0, 0)),
            pl.BlockSpec((1, D), lambda i: (0, 0)),
            pl.BlockSpec((1, D), lambda i: (0, 0)),
        ],
        out_specs=[
            pl.BlockSpec((R, D), lambda i: (i, 0)),
            pl.BlockSpec((R, D), lambda i: (i, 0)),
            pl.BlockSpec((R, D), lambda i: (i, 0)),
        ],
        out_shape=[
            jax.ShapeDtypeStruct((NP, D), _f32),
            jax.ShapeDtypeStruct((NC, D), _f32),
            jax.ShapeDtypeStruct((NP, D), _f32),
        ],
    )(pathway_h, cell_feat, W_a1[:D], W_a1[D:], W_g, W_proj,
      b_a1.reshape(1, D), b_g.reshape(1, D), b_proj.reshape(1, D))

    mean2, inv2 = pl.pallas_call(
        _tck2b_body,
        out_shape=[
            jax.ShapeDtypeStruct((1, NC), _f32),
            jax.ShapeDtypeStruct((1, NC), _f32),
        ],
    )(P)
    mean = mean2.reshape(NC)
    inv = inv2.reshape(NC)

    SA, SB, me, iv = _sck2(A, B, mean, inv, src, dst)

    RE = 1600
    ex2 = pl.pallas_call(
        _tck3_body,
        grid=(E // RE,),
        in_specs=[
            pl.BlockSpec((RE, D), lambda i: (i, 0)),
            pl.BlockSpec((RE, D), lambda i: (i, 0)),
            pl.BlockSpec((RE, 1), lambda i: (i, 0)),
            pl.BlockSpec((RE, 1), lambda i: (i, 0)),
            pl.BlockSpec((RE, 1), lambda i: (i, 0)),
            pl.BlockSpec((1, D), lambda i: (0, 0)),
            pl.BlockSpec((1, 1), lambda i: (0, 0)),
            pl.BlockSpec((1, 1), lambda i: (0, 0)),
        ],
        out_specs=pl.BlockSpec((RE, 1), lambda i: (i, 0)),
        out_shape=jax.ShapeDtypeStruct((E, 1), _f32),
    )(SA, SB, ew.reshape(E, 1), me.reshape(E, 1), iv.reshape(E, 1),
      W_a2.reshape(1, D), b_a2.reshape(1, 1), alpha_prior.reshape(1, 1))
    ex = ex2.reshape(E)

    M, den = _sck4(G, src, dst, ex)

    out = pl.pallas_call(
        _tck5_body,
        grid=(NC // R,),
        in_specs=[
            pl.BlockSpec((R, D), lambda i: (i, 0)),
            pl.BlockSpec((R, 1), lambda i: (i, 0)),
        ],
        out_specs=pl.BlockSpec((R, D), lambda i: (i, 0)),
        out_shape=jax.ShapeDtypeStruct((NC, D), _f32),
    )(M, den.reshape(NC, 1))
    return out


# speedup vs baseline: 3.1244x; 3.1244x over previous
import functools
import jax
import jax.numpy as jnp
from jax import lax
from jax.experimental import pallas as pl
from jax.experimental.pallas import tpu as pltpu
from jax.experimental.pallas import tpu_sc as plsc

NP = 10000
NC = 10000
E = 160000
D = 256

NCORES = 2
NSUB = 16
NW = NCORES * NSUB
C1 = E // NW
C4 = E // NSUB
HALF = NC // NCORES
TRASH = HALF

@functools.cache
def _mesh():
    return plsc.VectorSubcoreMesh(core_axis_name="c", subcore_axis_name="s",
                                  num_cores=NCORES, num_subcores=NSUB)

_f32 = jnp.float32
_i32 = jnp.int32


def _iota16():
    return lax.broadcasted_iota(_i32, (16,), 0)


def _sck1_body(dst_h, ew_h, cnt0_h, cnt1_h, s10_h, s11_h, s20_h, s21_h,
               acc_c, acc_1, acc_2, dstb2, ewb, onesb, ew2b, zbuf):
    c = lax.axis_index("c")
    s = lax.axis_index("s")
    wid = s * NCORES + c
    eb = pl.multiple_of(wid * C1, 8)

    z16f = jnp.zeros((16,), _f32)
    z16i = jnp.zeros((16,), _i32)

    @pl.when(s == 0)
    def _():
        def zb(i, _):
            zbuf[pl.ds(i * 16, 16)] = z16f
            return 0
        lax.fori_loop(0, NC // 16, zb, 0)
        pltpu.sync_copy(zbuf, acc_c)
        pltpu.sync_copy(zbuf, acc_1)
        pltpu.sync_copy(zbuf, acc_2)

    def fill(i, _):
        idx = i * 16 + _iota16()
        onesb[pl.ds(i * 16, 16)] = jnp.where(idx < C1, 1.0, 0.0).astype(_f32)
        return 0
    lax.fori_loop(0, 320, fill, 0)

    def ztail(i, _):
        ewb[pl.ds(C1 + i * 16, 16)] = z16f
        return 0
    lax.fori_loop(0, 120 // 16, ztail, 0)
    pltpu.sync_copy(ew_h.at[pl.ds(eb, C1)], ewb.at[pl.ds(0, C1)])

    def sq(i, _):
        v = ewb[pl.ds(i * 16, 16)]
        ew2b[pl.ds(i * 16, 16)] = v * v
        return 0
    lax.fori_loop(0, 320, sq, 0)

    for i in range(8):
        dstb2[39, pl.ds(i * 16, 16)] = z16i

    def ldidx(j, _):
        off = pl.multiple_of(j * 128, 128)
        pltpu.sync_copy(dst_h.at[pl.ds(eb + off, 128)], dstb2.at[j])
        return 0
    lax.fori_loop(0, 39, ldidx, 0)
    pltpu.sync_copy(dst_h.at[pl.ds(eb + 4992, 8)], dstb2.at[39, pl.ds(0, 8)])

    plsc.subcore_barrier()

    def scat(j, _):
        off = pl.multiple_of(j * 128, 128)
        idx = dstb2.at[j]
        pltpu.sync_copy(onesb.at[pl.ds(off, 128)], acc_c.at[idx], add=True)
        pltpu.sync_copy(ewb.at[pl.ds(off, 128)], acc_1.at[idx], add=True)
        pltpu.sync_copy(ew2b.at[pl.ds(off, 128)], acc_2.at[idx], add=True)
        return 0
    lax.fori_loop(0, 40, scat, 0)

    plsc.subcore_barrier()

    @pl.when((s == 0) & (c == 0))
    def _():
        pltpu.sync_copy(acc_c, cnt0_h)

    @pl.when((s == 0) & (c == 1))
    def _():
        pltpu.sync_copy(acc_c, cnt1_h)

    @pl.when((s == 1) & (c == 0))
    def _():
        pltpu.sync_copy(acc_1, s10_h)

    @pl.when((s == 1) & (c == 1))
    def _():
        pltpu.sync_copy(acc_1, s11_h)

    @pl.when((s == 2) & (c == 0))
    def _():
        pltpu.sync_copy(acc_2, s20_h)

    @pl.when((s == 2) & (c == 1))
    def _():
        pltpu.sync_copy(acc_2, s21_h)


@functools.cache
def _sck1():
  return functools.partial(
    pl.kernel,
    out_type=tuple(jax.ShapeDtypeStruct((NC,), _f32) for _ in range(6)),
    mesh=_mesh(),
    compiler_params=pltpu.CompilerParams(needs_layout_passes=False),
    scratch_types=[
        pltpu.VMEM_SHARED((NC,), _f32),
        pltpu.VMEM_SHARED((NC,), _f32),
        pltpu.VMEM_SHARED((NC,), _f32),
        pltpu.VMEM((40, 128), _i32),
        pltpu.VMEM((5120,), _f32),
        pltpu.VMEM((5120,), _f32),
        pltpu.VMEM((5120,), _f32),
        pltpu.VMEM((NC,), _f32),
    ],
)(_sck1_body)


def _sck2_body(a_h, b_h, mean_h, inv_h, src_h, dst_h,
               sa_h, sb_h, me_h, iv_h,
               srcb, dstb, bufA, bufB, bufm, bufi, s0, s1, s2, s3):
    c = lax.axis_index("c")
    s = lax.axis_index("s")
    wid = s * NCORES + c
    eb = pl.multiple_of(wid * C1, 8)

    pltpu.sync_copy(src_h.at[pl.ds(eb, C1)], srcb.at[pl.ds(0, C1)])
    pltpu.sync_copy(dst_h.at[pl.ds(eb, C1)], dstb.at[pl.ds(0, C1)])

    def blk(j, _):
        off = pl.multiple_of(jnp.minimum(j * 128, C1 - 128), 8)
        si = srcb.at[pl.ds(off, 128)]
        di = dstb.at[pl.ds(off, 128)]
        d1 = pltpu.async_copy(a_h.at[si], bufA, s0)
        d2 = pltpu.async_copy(b_h.at[di], bufB, s1)
        d3 = pltpu.async_copy(mean_h.at[di], bufm, s2)
        d4 = pltpu.async_copy(inv_h.at[di], bufi, s3)
        d1.wait()
        d2.wait()
        d3.wait()
        d4.wait()
        base = pl.multiple_of(eb + off, 8)
        pltpu.sync_copy(bufA, sa_h.at[pl.ds(base, 128)])
        pltpu.sync_copy(bufB, sb_h.at[pl.ds(base, 128)])
        pltpu.sync_copy(bufm, me_h.at[pl.ds(base, 128)])
        pltpu.sync_copy(bufi, iv_h.at[pl.ds(base, 128)])
        return 0
    lax.fori_loop(0, 40, blk, 0)


@functools.cache
def _sck2():
  return functools.partial(
    pl.kernel,
    out_type=(
        jax.ShapeDtypeStruct((E, D), _f32),
        jax.ShapeDtypeStruct((E, D), _f32),
        jax.ShapeDtypeStruct((E,), _f32),
        jax.ShapeDtypeStruct((E,), _f32),
    ),
    mesh=_mesh(),
    compiler_params=pltpu.CompilerParams(needs_layout_passes=False),
    scratch_types=[
        pltpu.VMEM((5120,), _i32),
        pltpu.VMEM((5120,), _i32),
        pltpu.VMEM((128, D), _f32),
        pltpu.VMEM((128, D), _f32),
        pltpu.VMEM((128,), _f32),
        pltpu.VMEM((128,), _f32),
        pltpu.SemaphoreType.DMA,
        pltpu.SemaphoreType.DMA,
        pltpu.SemaphoreType.DMA,
        pltpu.SemaphoreType.DMA,
    ],
)(_sck2_body)


CPT = 320
MROWS = NW * CPT
CH4 = 2000
NCH4 = E // CH4
CCAP = 3152
THR = 1024


def _sck4_body(g_h, src_h, dst_h, ex_h, m_h, den_h,
               accM, accD, srcb, dstb, exb, csrc, cdl, cev, bufG,
               sem_s, sem_d, sem_e, sem_g):
    c = lax.axis_index("c")
    s = lax.axis_index("s")
    wid = s * NCORES + c
    base_cell = wid * CPT

    z16f = jnp.zeros((16,), _f32)
    z16i = jnp.zeros((16,), _i32)

    def zacc(i, _):
        for k in range(16):
            accM[i, pl.ds(k * 16, 16)] = z16f
        accD[pl.ds(i * 16, 16)] = z16f
        return 0
    lax.fori_loop(0, 320, zacc, 0)

    def process(cnt):
        def pad(i, _):
            off = cnt + i * 16
            csrc[pl.ds(off, 16)] = z16i
            cdl[pl.ds(off, 16)] = z16i
            cev[pl.ds(off, 16)] = z16f
            return 0
        lax.fori_loop(0, 4, pad, 0)
        nblk = (cnt + 63) // 64

        def blk(i, _):
            boff = pl.multiple_of(i * 64, 8)
            pltpu.async_copy(g_h.at[csrc.at[pl.ds(boff, 64)]], bufG,
                             sem_g).wait()

            def rows(rr, _):
                dlv = cdl[pl.ds(boff + rr * 16, 16)]
                evv = cev[pl.ds(boff + rr * 16, 16)]
                for rloc in range(16):
                    sel = _iota16() == rloc
                    dlr = jnp.sum(jnp.where(sel, dlv, 0))
                    evr = jnp.sum(jnp.where(sel, evv, 0.0))
                    evb = lax.broadcast_in_dim(evr, (16,), ())
                    r = rr * 16 + rloc
                    for k in range(16):
                        accM[dlr, pl.ds(k * 16, 16)] = (
                            accM[dlr, pl.ds(k * 16, 16)]
                            + evb * bufG[r, pl.ds(k * 16, 16)])
                    dbase = dlr * 16
                    accD[pl.ds(dbase, 16)] = (
                        accD[pl.ds(dbase, 16)]
                        + jnp.where(_iota16() == 0, evr, 0.0))
                return 0
            lax.fori_loop(0, 4, rows, 0)
            return 0
        lax.fori_loop(0, nblk, blk, 0)
        return 0

    def chunk(ch, cnt):
        off = pl.multiple_of(ch * CH4, 8)
        d1 = pltpu.async_copy(src_h.at[pl.ds(off, CH4)], srcb, sem_s)
        d2 = pltpu.async_copy(dst_h.at[pl.ds(off, CH4)], dstb, sem_d)
        d3 = pltpu.async_copy(ex_h.at[pl.ds(off, CH4)], exb, sem_e)
        d1.wait()
        d2.wait()
        d3.wait()

        def scan(v, cnt):
            sv = srcb[pl.ds(v * 16, 16)]
            dv = dstb[pl.ds(v * 16, 16)]
            ev = exb[pl.ds(v * 16, 16)]
            dl = dv - base_cell
            m = (dl >= 0) & (dl < CPT)
            plsc.store_compressed(csrc.at[pl.ds(cnt, 16)], sv, mask=m)
            plsc.store_compressed(cdl.at[pl.ds(cnt, 16)], dl, mask=m)
            plsc.store_compressed(cev.at[pl.ds(cnt, 16)], ev, mask=m)
            return cnt + jnp.sum(m.astype(_i32))
        cnt = lax.fori_loop(0, CH4 // 16, scan, cnt)

        def flush(cnt):
            process(cnt)
            return jnp.int32(0)
        cnt = lax.cond(cnt >= THR, flush, lambda cnt: cnt, cnt)
        return cnt

    cnt = lax.fori_loop(0, NCH4, chunk, jnp.int32(0))

    @pl.when(cnt > 0)
    def _():
        process(cnt)

    ob = pl.multiple_of(wid * CPT, 8)
    pltpu.sync_copy(accM.at[pl.ds(0, CPT)], m_h.at[pl.ds(ob, CPT)])
    db = pl.multiple_of(wid * 5120, 8)
    pltpu.sync_copy(accD, den_h.at[pl.ds(db, 5120)])


@functools.cache
def _sck4():
  return functools.partial(
    pl.kernel,
    out_type=(
        jax.ShapeDtypeStruct((MROWS, D), _f32),
        jax.ShapeDtypeStruct((NW * 5120,), _f32),
    ),
    mesh=_mesh(),
    compiler_params=pltpu.CompilerParams(needs_layout_passes=False),
    scratch_types=[
        pltpu.VMEM((320, D), _f32),
        pltpu.VMEM((5120,), _f32),
        pltpu.VMEM((CH4,), _i32),
        pltpu.VMEM((CH4,), _i32),
        pltpu.VMEM((CH4,), _f32),
        pltpu.VMEM((CCAP,), _i32),
        pltpu.VMEM((CCAP,), _i32),
        pltpu.VMEM((CCAP,), _f32),
        pltpu.VMEM((64, D), _f32),
        pltpu.SemaphoreType.DMA,
        pltpu.SemaphoreType.DMA,
        pltpu.SemaphoreType.DMA,
        pltpu.SemaphoreType.DMA,
    ],
)(_sck4_body)


def _tck2a_body(ph, cf, w1p, w1c, wg, wp, b1, bg, bp, a_o, b_o, g_o):
    x = ph[...]
    a_o[...] = jnp.dot(x, w1p[...], preferred_element_type=_f32)
    b_o[...] = jnp.dot(cf[...], w1c[...], preferred_element_type=_f32) + b1[...]
    gate = jax.nn.sigmoid(jnp.dot(x, wg[...], preferred_element_type=_f32) + bg[...])
    g_o[...] = gate * (jnp.dot(x, wp[...], preferred_element_type=_f32) + bp[...])


def _tck2b_body(c0, c1, a0, a1, b0, b1, mean_o, inv_o):
    cnt = c0[...] + c1[...]
    s1 = a0[...] + a1[...]
    s2 = b0[...] + b1[...]
    cntc = jnp.maximum(cnt, 1.0)
    mean = s1 / cntc
    var = jnp.maximum(s2 / cntc - mean * mean, 0.0)
    mean_o[...] = mean
    inv_o[...] = 1.0 / (jnp.sqrt(var + 1e-6) + 1e-6)


def _tck3_body(sa, sb, ew, me, iv, w2, b2, al, ex_o):
    t = jnp.tanh(sa[...] + sb[...])
    att = jnp.sum(t * w2[...], axis=1, keepdims=True)
    att = att + b2[...] + al[...] * (ew[...] - me[...]) * iv[...]
    ex_o[...] = jnp.exp(att)


def _tck5_body(m, den, out):
    out[...] = m[...] / (den[...] + 1e-16)


def kernel(pathway_h, cell_feat, edge_index, edge_weight,
           W_proj, b_proj, W_a1, b_a1, W_a2, b_a2, W_g, b_g, alpha_prior):
    src = edge_index[0].astype(_i32)
    dst = edge_index[1].astype(_i32)
    ew = edge_weight.astype(_f32)

    cnt0, cnt1, s10, s11, s20, s21 = _sck1()(dst, ew)

    R = 1000
    A, B, G = pl.pallas_call(
        _tck2a_body,
        grid=(NP // R,),
        in_specs=[
            pl.BlockSpec((R, D), lambda i: (i, 0)),
            pl.BlockSpec((R, D), lambda i: (i, 0)),
            pl.BlockSpec((D, D), lambda i: (0, 0)),
            pl.BlockSpec((D, D), lambda i: (0, 0)),
            pl.BlockSpec((D, D), lambda i: (0, 0)),
            pl.BlockSpec((D, D), lambda i: (0, 0)),
            pl.BlockSpec((1, D), lambda i: (0, 0)),
            pl.BlockSpec((1, D), lambda i: (0, 0)),
            pl.BlockSpec((1, D), lambda i: (0, 0)),
        ],
        out_specs=[
            pl.BlockSpec((R, D), lambda i: (i, 0)),
            pl.BlockSpec((R, D), lambda i: (i, 0)),
            pl.BlockSpec((R, D), lambda i: (i, 0)),
        ],
        out_shape=[
            jax.ShapeDtypeStruct((NP, D), _f32),
            jax.ShapeDtypeStruct((NC, D), _f32),
            jax.ShapeDtypeStruct((NP, D), _f32),
        ],
    )(pathway_h, cell_feat, W_a1[:D], W_a1[D:], W_g, W_proj,
      b_a1.reshape(1, D), b_g.reshape(1, D), b_proj.reshape(1, D))

    mean2, inv2 = pl.pallas_call(
        _tck2b_body,
        out_shape=[
            jax.ShapeDtypeStruct((1, NC), _f32),
            jax.ShapeDtypeStruct((1, NC), _f32),
        ],
    )(cnt0.reshape(1, NC), cnt1.reshape(1, NC), s10.reshape(1, NC),
      s11.reshape(1, NC), s20.reshape(1, NC), s21.reshape(1, NC))
    mean = mean2.reshape(NC)
    inv = inv2.reshape(NC)

    SA, SB, me, iv = _sck2()(A, B, mean, inv, src, dst)

    RE = 1600
    ex2 = pl.pallas_call(
        _tck3_body,
        grid=(E // RE,),
        in_specs=[
            pl.BlockSpec((RE, D), lambda i: (i, 0)),
            pl.BlockSpec((RE, D), lambda i: (i, 0)),
            pl.BlockSpec((RE, 1), lambda i: (i, 0)),
            pl.BlockSpec((RE, 1), lambda i: (i, 0)),
            pl.BlockSpec((RE, 1), lambda i: (i, 0)),
            pl.BlockSpec((1, D), lambda i: (0, 0)),
            pl.BlockSpec((1, 1), lambda i: (0, 0)),
            pl.BlockSpec((1, 1), lambda i: (0, 0)),
        ],
        out_specs=pl.BlockSpec((RE, 1), lambda i: (i, 0)),
        out_shape=jax.ShapeDtypeStruct((E, 1), _f32),
    )(SA, SB, ew.reshape(E, 1), me.reshape(E, 1), iv.reshape(E, 1),
      W_a2.reshape(1, D), b_a2.reshape(1, 1), alpha_prior.reshape(1, 1))
    ex = ex2.reshape(E)

    Mp, denp = _sck4()(G, src, dst, ex)
    M = Mp[:NC]
    den = denp.reshape(NW, 320, 16)[:, :CPT, 0].reshape(-1)[:NC]

    out = pl.pallas_call(
        _tck5_body,
        grid=(NC // R,),
        in_specs=[
            pl.BlockSpec((R, D), lambda i: (i, 0)),
            pl.BlockSpec((R, 1), lambda i: (i, 0)),
        ],
        out_specs=pl.BlockSpec((R, D), lambda i: (i, 0)),
        out_shape=jax.ShapeDtypeStruct((NC, D), _f32),
    )(M, den.reshape(NC, 1))
    return out
